# baseline (speedup 1.0000x reference)
import functools

import jax
import jax.numpy as jnp
from jax import lax
from jax.experimental import pallas as pl
from jax.experimental.pallas import tpu as pltpu
from jax.experimental.pallas import tpu_sc as plsc

_HIDDEN = 64
_NW = 32
_NBUF = 4
_LOOK = 2


@jax.jit
def _embed_gather(ids_in, table):
    batch, seq = ids_in.shape
    nb = batch // _NW
    h1 = 104
    h2 = seq - h1
    mesh = plsc.VectorSubcoreMesh(core_axis_name="c", subcore_axis_name="s")

    @functools.partial(
        pl.kernel,
        out_type=jax.ShapeDtypeStruct((batch, seq, _HIDDEN), jnp.float32),
        mesh=mesh,
        scratch_types=[
            pltpu.VMEM((nb, seq), jnp.int32),
            pltpu.VMEM((_NBUF, seq, _HIDDEN), jnp.float32),
            pltpu.SemaphoreType.DMA((_NBUF,)),
            pltpu.SemaphoreType.DMA((_NBUF,)),
        ],
        compiler_params=pltpu.CompilerParams(use_tc_tiling_on_sc=False),
    )
    def k(ids_hbm, table_hbm, out_hbm, idx_v, rows_v, gsem, ssem):
        wid = lax.axis_index("s") * 2 + lax.axis_index("c")
        b0 = wid * nb
        pltpu.sync_copy(ids_hbm.at[pl.ds(b0, nb), :], idx_v)

        def gather_descs(j, b):
            return (
                pltpu.make_async_copy(
                    table_hbm.at[idx_v.at[j, pl.ds(0, h1)]],
                    rows_v.at[b, pl.ds(0, h1)],
                    gsem.at[b],
                ),
                pltpu.make_async_copy(
                    table_hbm.at[idx_v.at[j, pl.ds(h1, h2)]],
                    rows_v.at[b, pl.ds(h1, h2)],
                    gsem.at[b],
                ),
            )

        def store_desc(j, b):
            return pltpu.make_async_copy(
                rows_v.at[b], out_hbm.at[b0 + j], ssem.at[b])

        for j in range(_LOOK):
            for d in gather_descs(j, j % _NBUF):
                d.start()

        def slot(j, b, first, last):
            for d in gather_descs(j, b):
                d.wait()
            store_desc(j, b).start()
            jn = j + _LOOK
            bn = (b + _LOOK) % _NBUF
            if not first:
                store_desc(jn - _NBUF, bn).wait()
            if not last:
                for d in gather_descs(jn, bn):
                    d.start()

        for b in range(_NBUF):
            slot(b, b, first=(b < _LOOK), last=False)

        @pl.loop(_NBUF, nb - _NBUF, step=_NBUF)
        def _(g):
            for b in range(_NBUF):
                slot(g + b, b, first=False, last=False)

        g_last = nb - _NBUF
        for b in range(_NBUF):
            slot(g_last + b, b, first=False, last=(b >= _NBUF - _LOOK))

        for b in range(_NBUF - _LOOK, _NBUF):
            store_desc(g_last + b, b).wait()

    return k(ids_in, table)


def kernel(input_ids, table):
    return _embed_gather(input_ids, table)
